# baseline (speedup 1.0000x reference)
import functools

import jax
import jax.numpy as jnp
from jax import lax
from jax.experimental import pallas as pl
from jax.experimental.pallas import tpu as pltpu
from jax.experimental.pallas import tpu_sc as plsc

N_OBJ = 2048
NUM_CLS = 151
HIDDEN = 256
P = 131072
NN = N_OBJ * N_OBJ

NW = 32
PPW = P // NW
ZCH = NN // 16


def _tc_vals_body(lg_ref, ws_ref, bs_ref, wo_ref, bo_ref, wcs_ref, wco_ref,
                  bc_ref, sval_ref, oval_ref):
    x = lg_ref[...]
    m = jnp.max(x, axis=1, keepdims=True)
    e = jnp.exp(x - m)
    p = e / jnp.sum(e, axis=1, keepdims=True)
    hs = jnp.maximum(
        jnp.dot(p, ws_ref[...], preferred_element_type=jnp.float32) + bs_ref[...], 0.0)
    ho = jnp.maximum(
        jnp.dot(p, wo_ref[...], preferred_element_type=jnp.float32) + bo_ref[...], 0.0)
    sval_ref[...] = jnp.sum(hs * wcs_ref[...], axis=1, keepdims=True) + bc_ref[0, 0]
    oval_ref[...] = jnp.sum(ho * wco_ref[...], axis=1, keepdims=True)


_tc_vals = pl.pallas_call(
    _tc_vals_body,
    out_shape=(jax.ShapeDtypeStruct((N_OBJ, 1), jnp.float32),
               jax.ShapeDtypeStruct((N_OBJ, 1), jnp.float32)),
)


def _tc_zeros_body(mat_hbm, zbuf, sem):
    zbuf[...] = jnp.zeros_like(zbuf)
    copies = [
        pltpu.async_copy(zbuf, mat_hbm.at[pl.ds(k * ZCH, ZCH)], sem)
        for k in range(NN // ZCH)
    ]
    for d in copies:
        d.wait()


_tc_zeros_into = functools.partial(
    pl.kernel,
    out_type=(),
    mesh=pltpu.create_tensorcore_mesh("tc"),
    scratch_types=(
        pltpu.VMEM((ZCH,), jnp.float32),
        pltpu.SemaphoreType.DMA,
    ),
)(_tc_zeros_body)


def _sc_body(pairs_hbm, sval_hbm, oval_hbm, mat_hbm, logits_hbm,
             pair_v, stab_v, otab_v, logit_v, score_v, fidx_v, stsem, ssem):
    c = lax.axis_index("c")
    s = lax.axis_index("s")
    w = c * 16 + s

    with jax.named_scope("stage"):
        d1 = pltpu.async_copy(
            pairs_hbm.at[pl.ds(w * PPW, PPW), :], pair_v, stsem)
        d2 = pltpu.async_copy(sval_hbm, stab_v, stsem)
        d3 = pltpu.async_copy(oval_hbm, otab_v, stsem)
        d1.wait()
        d2.wait()
        d3.wait()

    def body(m, carry):
        lane = lax.iota(jnp.int32, 16)
        rows = m * 16 + lane
        zero16 = jnp.zeros((16,), jnp.int32)
        ii = plsc.load_gather(pair_v, [rows, zero16])
        jj = plsc.load_gather(pair_v, [rows, zero16 + 1])
        sv = plsc.load_gather(stab_v, [ii])
        ov = plsc.load_gather(otab_v, [jj])
        lg = sv + ov
        logit_v[pl.ds(m * 16, 16)] = lg
        sc = 1.0 / (1.0 + jnp.exp(-lg))
        score_v[pl.ds(m * 16, 16)] = sc
        fidx_v[pl.ds(m * 16, 16)] = ii * N_OBJ + jj
        return carry
    with jax.named_scope("compute"):
        lax.fori_loop(0, PPW // 16, body, 0)

    with jax.named_scope("logits_out"):
        pltpu.sync_copy(logit_v, logits_hbm.at[pl.ds(w * PPW, PPW)])

    with jax.named_scope("scatter"):
        pltpu.async_copy(score_v, mat_hbm.at[fidx_v], ssem).wait()


_sc_scatter = functools.partial(
    pl.kernel,
    out_type=jax.ShapeDtypeStruct((P,), jnp.float32),
    mesh=plsc.VectorSubcoreMesh(core_axis_name="c", subcore_axis_name="s"),
    compiler_params=pltpu.CompilerParams(
        needs_layout_passes=False, use_tc_tiling_on_sc=False),
    scratch_types=(
        pltpu.VMEM((PPW, 2), jnp.int32),
        pltpu.VMEM((N_OBJ,), jnp.float32),
        pltpu.VMEM((N_OBJ,), jnp.float32),
        pltpu.VMEM((PPW,), jnp.float32),
        pltpu.VMEM((PPW,), jnp.float32),
        pltpu.VMEM((PPW,), jnp.int32),
        pltpu.SemaphoreType.DMA,
        pltpu.SemaphoreType.DMA,
    ),
)(_sc_body)


def kernel(visual_feat, pred_logits, pair_idx, W_sub, b_sub, W_obj, b_obj,
           W_cls, b_cls):
    del visual_feat
    ws_cls = W_cls[:HIDDEN].reshape(1, HIDDEN)
    wo_cls = W_cls[HIDDEN:].reshape(1, HIDDEN)
    sval, oval = _tc_vals(pred_logits, W_sub, b_sub.reshape(1, HIDDEN),
                          W_obj, b_obj.reshape(1, HIDDEN),
                          ws_cls, wo_cls, b_cls.reshape(1, 1))
    mat_ref = jax.new_ref(jnp.empty((NN,), jnp.float32))
    _tc_zeros_into(mat_ref)
    logits = _sc_scatter(pair_idx, sval.reshape(-1), oval.reshape(-1), mat_ref)
    return logits, mat_ref[...].reshape(N_OBJ, N_OBJ)


# speedup vs baseline: 1.2067x; 1.2067x over previous
import functools

import jax
import jax.numpy as jnp
from jax import lax
from jax.experimental import pallas as pl
from jax.experimental.pallas import tpu as pltpu
from jax.experimental.pallas import tpu_sc as plsc

N_OBJ = 2048
NUM_CLS = 151
HIDDEN = 256
P = 131072
NN = N_OBJ * N_OBJ

NW = 32
PPW = P // NW
PROWS = PPW * 2 // 128
NCH = 4
CH = PPW // NCH


def _tc_vals_body(lg_ref, ws_ref, bs_ref, wo_ref, bo_ref, wcs_ref, wco_ref,
                  bc_ref, sval_ref, oval_ref):
    x = lg_ref[...]
    m = jnp.max(x, axis=1, keepdims=True)
    e = jnp.exp(x - m)
    p = e / jnp.sum(e, axis=1, keepdims=True)
    hs = jnp.maximum(
        jnp.dot(p, ws_ref[...], preferred_element_type=jnp.float32) + bs_ref[...], 0.0)
    ho = jnp.maximum(
        jnp.dot(p, wo_ref[...], preferred_element_type=jnp.float32) + bo_ref[...], 0.0)
    sval_ref[...] = jnp.sum(hs * wcs_ref[...], axis=1, keepdims=True) + bc_ref[0, 0]
    oval_ref[...] = jnp.sum(ho * wco_ref[...], axis=1, keepdims=True)


_tc_vals = pl.pallas_call(
    _tc_vals_body,
    out_shape=(jax.ShapeDtypeStruct((N_OBJ, 1), jnp.float32),
               jax.ShapeDtypeStruct((N_OBJ, 1), jnp.float32)),
)


def _sc_body(pairs_hbm, sval_hbm, oval_hbm, mat_hbm, logits_hbm,
             pair_v, stab_v, otab_v, logit_v,
             score0, score1, score2, score3,
             fidx0, fidx1, fidx2, fidx3, stsem, ssem, lsem):
    scores = (score0, score1, score2, score3)
    fidxs = (fidx0, fidx1, fidx2, fidx3)
    c = lax.axis_index("c")
    s = lax.axis_index("s")
    w = c * 16 + s

    d1 = pltpu.async_copy(
        pairs_hbm.at[pl.ds(w * PROWS, PROWS), :], pair_v, stsem)
    d2 = pltpu.async_copy(sval_hbm, stab_v, stsem)
    d3 = pltpu.async_copy(oval_hbm, otab_v, stsem)
    d1.wait()
    d2.wait()
    d3.wait()

    zero16 = jnp.zeros((16,), jnp.int32)

    def make_body(score_c, fidx_c, ch):
        def body(m, carry):
            lane = lax.iota(jnp.int32, 16)
            flat = m * 32 + 2 * lane
            ii = plsc.load_gather(pair_v, [flat // 128, flat % 128])
            jj = plsc.load_gather(pair_v, [(flat + 1) // 128, (flat + 1) % 128])
            sv = plsc.load_gather(stab_v, [ii])
            ov = plsc.load_gather(otab_v, [jj])
            lg = sv + ov
            logit_v[pl.ds(m * 16, 16)] = lg
            sc = 1.0 / (1.0 + jnp.exp(-lg))
            o = m * 16 - ch * CH
            score_c[pl.ds(o, 16)] = sc
            fidx_c[pl.ds(o, 16)] = ii * N_OBJ + jj
            return carry
        return body

    scat = []
    for ch in range(NCH):
        lax.fori_loop(ch * (CH // 16), (ch + 1) * (CH // 16),
                      make_body(scores[ch], fidxs[ch], ch), 0)
        scat.append(
            pltpu.async_copy(scores[ch], mat_hbm.at[fidxs[ch]], ssem))

    dl = pltpu.async_copy(logit_v, logits_hbm.at[pl.ds(w * PPW, PPW)], lsem)
    for d in scat:
        d.wait()
    dl.wait()


_sc_scatter = functools.partial(
    pl.kernel,
    out_type=jax.ShapeDtypeStruct((P,), jnp.float32),
    mesh=plsc.VectorSubcoreMesh(core_axis_name="c", subcore_axis_name="s"),
    compiler_params=pltpu.CompilerParams(needs_layout_passes=False),
    scratch_types=(
        pltpu.VMEM((PROWS, 128), jnp.int32),
        pltpu.VMEM((N_OBJ,), jnp.float32),
        pltpu.VMEM((N_OBJ,), jnp.float32),
        pltpu.VMEM((PPW,), jnp.float32),
        pltpu.VMEM((CH,), jnp.float32),
        pltpu.VMEM((CH,), jnp.float32),
        pltpu.VMEM((CH,), jnp.float32),
        pltpu.VMEM((CH,), jnp.float32),
        pltpu.VMEM((CH,), jnp.int32),
        pltpu.VMEM((CH,), jnp.int32),
        pltpu.VMEM((CH,), jnp.int32),
        pltpu.VMEM((CH,), jnp.int32),
        pltpu.SemaphoreType.DMA,
        pltpu.SemaphoreType.DMA,
        pltpu.SemaphoreType.DMA,
    ),
)(_sc_body)


def kernel(visual_feat, pred_logits, pair_idx, W_sub, b_sub, W_obj, b_obj,
           W_cls, b_cls):
    del visual_feat
    ws_cls = W_cls[:HIDDEN].reshape(1, HIDDEN)
    wo_cls = W_cls[HIDDEN:].reshape(1, HIDDEN)
    sval, oval = _tc_vals(pred_logits, W_sub, b_sub.reshape(1, HIDDEN),
                          W_obj, b_obj.reshape(1, HIDDEN),
                          ws_cls, wo_cls, b_cls.reshape(1, 1))
    mat_ref = jax.new_ref(jnp.zeros((NN,), jnp.float32))
    logits = _sc_scatter(pair_idx.reshape(P * 2 // 128, 128),
                         sval.reshape(-1), oval.reshape(-1), mat_ref)
    return logits, mat_ref[...].reshape(N_OBJ, N_OBJ)


# speedup vs baseline: 1.2081x; 1.0012x over previous
import functools

import jax
import jax.numpy as jnp
from jax import lax
from jax.experimental import pallas as pl
from jax.experimental.pallas import tpu as pltpu
from jax.experimental.pallas import tpu_sc as plsc

N_OBJ = 2048
NUM_CLS = 151
HIDDEN = 256
P = 131072
NN = N_OBJ * N_OBJ

NW = 32
PPW = P // NW
PROWS = PPW * 2 // 128
NCH = 4
CH = PPW // NCH


def _tc_vals_body(lg_ref, ws_ref, bs_ref, wo_ref, bo_ref, wcs_ref, wco_ref,
                  bc_ref, sval_ref, oval_ref):
    x = lg_ref[...]
    m = jnp.max(x, axis=1, keepdims=True)
    e = jnp.exp(x - m)
    p = e / jnp.sum(e, axis=1, keepdims=True)
    hs = jnp.maximum(
        jnp.dot(p, ws_ref[...], preferred_element_type=jnp.float32) + bs_ref[...], 0.0)
    ho = jnp.maximum(
        jnp.dot(p, wo_ref[...], preferred_element_type=jnp.float32) + bo_ref[...], 0.0)
    sval_ref[...] = jnp.sum(hs * wcs_ref[...], axis=1, keepdims=True) + bc_ref[0, 0]
    oval_ref[...] = jnp.sum(ho * wco_ref[...], axis=1, keepdims=True)


_tc_vals = pl.pallas_call(
    _tc_vals_body,
    out_shape=(jax.ShapeDtypeStruct((N_OBJ, 1), jnp.float32),
               jax.ShapeDtypeStruct((N_OBJ, 1), jnp.float32)),
)


def _sc_body(pairs_hbm, sval_hbm, oval_hbm, mat_hbm, logits_hbm,
             pair_v, stab_v, otab_v, logit_v,
             score0, score1, score2, score3,
             fidx0, fidx1, fidx2, fidx3, stsem, ssem, lsem):
    scores = (score0, score1, score2, score3)
    fidxs = (fidx0, fidx1, fidx2, fidx3)
    c = lax.axis_index("c")
    s = lax.axis_index("s")
    w = c * 16 + s

    d1 = pltpu.async_copy(
        pairs_hbm.at[pl.ds(w * PROWS, PROWS), :], pair_v, stsem)
    d2 = pltpu.async_copy(sval_hbm, stab_v, stsem)
    d3 = pltpu.async_copy(oval_hbm, otab_v, stsem)
    d1.wait()
    d2.wait()
    d3.wait()

    zero16 = jnp.zeros((16,), jnp.int32)

    def make_body(score_c, fidx_c, ch):
        def body(m, carry):
            lane = lax.iota(jnp.int32, 16)
            flat = m * 32 + 2 * lane
            ii = plsc.load_gather(pair_v, [flat // 128, flat % 128])
            jj = plsc.load_gather(pair_v, [(flat + 1) // 128, (flat + 1) % 128])
            sv = plsc.load_gather(stab_v, [ii])
            ov = plsc.load_gather(otab_v, [jj])
            lg = sv + ov
            logit_v[pl.ds(m * 16, 16)] = lg
            sc = 1.0 / (1.0 + jnp.exp(-lg))
            o = m * 16 - ch * CH
            score_c[pl.ds(o, 16)] = sc
            fidx_c[pl.ds(o, 16)] = ii * N_OBJ + jj
            return carry
        return body

    scat = []
    for ch in range(NCH):
        lax.fori_loop(ch * (CH // 16), (ch + 1) * (CH // 16),
                      make_body(scores[ch], fidxs[ch], ch), 0)
        scat.append(
            pltpu.async_copy(scores[ch], mat_hbm.at[fidxs[ch]], ssem))

    dl = pltpu.async_copy(logit_v, logits_hbm.at[pl.ds(w * PPW, PPW)], lsem)
    for d in scat:
        d.wait()
    dl.wait()


_sc_scatter = functools.partial(
    pl.kernel,
    out_type=jax.ShapeDtypeStruct((P,), jnp.float32),
    mesh=plsc.VectorSubcoreMesh(core_axis_name="c", subcore_axis_name="s"),
    compiler_params=pltpu.CompilerParams(
        needs_layout_passes=False,
        disable_bounds_checks=True,
        disable_semaphore_checks=True,
        skip_device_barrier=True,
    ),
    scratch_types=(
        pltpu.VMEM((PROWS, 128), jnp.int32),
        pltpu.VMEM((N_OBJ,), jnp.float32),
        pltpu.VMEM((N_OBJ,), jnp.float32),
        pltpu.VMEM((PPW,), jnp.float32),
        pltpu.VMEM((CH,), jnp.float32),
        pltpu.VMEM((CH,), jnp.float32),
        pltpu.VMEM((CH,), jnp.float32),
        pltpu.VMEM((CH,), jnp.float32),
        pltpu.VMEM((CH,), jnp.int32),
        pltpu.VMEM((CH,), jnp.int32),
        pltpu.VMEM((CH,), jnp.int32),
        pltpu.VMEM((CH,), jnp.int32),
        pltpu.SemaphoreType.DMA,
        pltpu.SemaphoreType.DMA,
        pltpu.SemaphoreType.DMA,
    ),
)(_sc_body)


def kernel(visual_feat, pred_logits, pair_idx, W_sub, b_sub, W_obj, b_obj,
           W_cls, b_cls):
    del visual_feat
    ws_cls = W_cls[:HIDDEN].reshape(1, HIDDEN)
    wo_cls = W_cls[HIDDEN:].reshape(1, HIDDEN)
    sval, oval = _tc_vals(pred_logits, W_sub, b_sub.reshape(1, HIDDEN),
                          W_obj, b_obj.reshape(1, HIDDEN),
                          ws_cls, wo_cls, b_cls.reshape(1, 1))
    mat_ref = jax.new_ref(jnp.zeros((NN,), jnp.float32))
    logits = _sc_scatter(pair_idx.reshape(P * 2 // 128, 128),
                         sval.reshape(-1), oval.reshape(-1), mat_ref)
    return logits, mat_ref[...].reshape(N_OBJ, N_OBJ)


# speedup vs baseline: 1.2148x; 1.0055x over previous
import functools

import jax
import jax.numpy as jnp
from jax import lax
from jax.experimental import pallas as pl
from jax.experimental.pallas import tpu as pltpu
from jax.experimental.pallas import tpu_sc as plsc

N_OBJ = 2048
NUM_CLS = 151
HIDDEN = 256
P = 131072
NN = N_OBJ * N_OBJ

NW = 32
PPW = P // NW
PROWS = PPW * 2 // 128
NCH = 4
CH = PPW // NCH


def _tc_vals_body(lg_ref, ws_ref, bs_ref, wo_ref, bo_ref, wcs_ref, wco_ref,
                  bc_ref, sval_ref, oval_ref):
    x = lg_ref[...]
    m = jnp.max(x, axis=1, keepdims=True)
    e = jnp.exp(x - m)
    p = e / jnp.sum(e, axis=1, keepdims=True)
    hs = jnp.maximum(
        jnp.dot(p, ws_ref[...], preferred_element_type=jnp.float32) + bs_ref[...], 0.0)
    ho = jnp.maximum(
        jnp.dot(p, wo_ref[...], preferred_element_type=jnp.float32) + bo_ref[...], 0.0)
    sval_ref[...] = jnp.sum(hs * wcs_ref[...], axis=1) + bc_ref[0, 0]
    oval_ref[...] = jnp.sum(ho * wco_ref[...], axis=1)


_tc_vals = pl.pallas_call(
    _tc_vals_body,
    out_shape=(jax.ShapeDtypeStruct((N_OBJ,), jnp.float32),
               jax.ShapeDtypeStruct((N_OBJ,), jnp.float32)),
)


def _sc_body(pairs_hbm, sval_hbm, oval_hbm, mat_hbm, logits_hbm,
             pair_v, stab_v, otab_v, logit_v,
             score0, score1, score2, score3,
             fidx0, fidx1, fidx2, fidx3, stsem, ssem, lsem):
    scores = (score0, score1, score2, score3)
    fidxs = (fidx0, fidx1, fidx2, fidx3)
    c = lax.axis_index("c")
    s = lax.axis_index("s")
    w = c * 16 + s

    d1 = pltpu.async_copy(
        pairs_hbm.at[pl.ds(w * PROWS, PROWS), :], pair_v, stsem)
    d2 = pltpu.async_copy(sval_hbm, stab_v, stsem)
    d3 = pltpu.async_copy(oval_hbm, otab_v, stsem)
    d1.wait()
    d2.wait()
    d3.wait()

    zero16 = jnp.zeros((16,), jnp.int32)

    def make_body(score_c, fidx_c, ch):
        def body(m, carry):
            lane = lax.iota(jnp.int32, 16)
            flat = m * 32 + 2 * lane
            ii = plsc.load_gather(pair_v, [flat // 128, flat % 128])
            jj = plsc.load_gather(pair_v, [(flat + 1) // 128, (flat + 1) % 128])
            sv = plsc.load_gather(stab_v, [ii])
            ov = plsc.load_gather(otab_v, [jj])
            lg = sv + ov
            logit_v[pl.ds(m * 16, 16)] = lg
            sc = 1.0 / (1.0 + jnp.exp(-lg))
            o = m * 16 - ch * CH
            score_c[pl.ds(o, 16)] = sc
            fidx_c[pl.ds(o, 16)] = ii * N_OBJ + jj
            return carry
        return body

    scat = []
    for ch in range(NCH):
        lax.fori_loop(ch * (CH // 16), (ch + 1) * (CH // 16),
                      make_body(scores[ch], fidxs[ch], ch), 0)
        scat.append(
            pltpu.async_copy(scores[ch], mat_hbm.at[fidxs[ch]], ssem))

    dl = pltpu.async_copy(logit_v, logits_hbm.at[pl.ds(w * PPW, PPW)], lsem)
    for d in scat:
        d.wait()
    dl.wait()


_sc_scatter = functools.partial(
    pl.kernel,
    out_type=jax.ShapeDtypeStruct((P,), jnp.float32),
    mesh=plsc.VectorSubcoreMesh(core_axis_name="c", subcore_axis_name="s"),
    compiler_params=pltpu.CompilerParams(
        needs_layout_passes=False,
        disable_bounds_checks=True,
        disable_semaphore_checks=True,
        skip_device_barrier=True,
    ),
    scratch_types=(
        pltpu.VMEM((PROWS, 128), jnp.int32),
        pltpu.VMEM((N_OBJ,), jnp.float32),
        pltpu.VMEM((N_OBJ,), jnp.float32),
        pltpu.VMEM((PPW,), jnp.float32),
        pltpu.VMEM((CH,), jnp.float32),
        pltpu.VMEM((CH,), jnp.float32),
        pltpu.VMEM((CH,), jnp.float32),
        pltpu.VMEM((CH,), jnp.float32),
        pltpu.VMEM((CH,), jnp.int32),
        pltpu.VMEM((CH,), jnp.int32),
        pltpu.VMEM((CH,), jnp.int32),
        pltpu.VMEM((CH,), jnp.int32),
        pltpu.SemaphoreType.DMA,
        pltpu.SemaphoreType.DMA,
        pltpu.SemaphoreType.DMA,
    ),
)(_sc_body)


def kernel(visual_feat, pred_logits, pair_idx, W_sub, b_sub, W_obj, b_obj,
           W_cls, b_cls):
    del visual_feat
    ws_cls = W_cls[:HIDDEN].reshape(1, HIDDEN)
    wo_cls = W_cls[HIDDEN:].reshape(1, HIDDEN)
    sval, oval = _tc_vals(pred_logits, W_sub, b_sub.reshape(1, HIDDEN),
                          W_obj, b_obj.reshape(1, HIDDEN),
                          ws_cls, wo_cls, b_cls.reshape(1, 1))
    mat_ref = jax.new_ref(jnp.zeros((NN,), jnp.float32))
    logits = _sc_scatter(pair_idx.reshape(P * 2 // 128, 128), sval, oval,
                         mat_ref)
    return logits, mat_ref[...].reshape(N_OBJ, N_OBJ)
